# initial kernel scaffold (unmeasured)
import jax
import jax.numpy as jnp
from jax import lax
from jax.experimental import pallas as pl
from jax.experimental.pallas import tpu as pltpu


def kernel(Q, K, V, bt, lens):
    B, QL, H, D = Q.shape
    P_loc, BS, _, _ = K.shape
    NB = bt.shape[1]
    KT = P_loc * BS
    scale = D ** -0.5

    def body(q_ref, k_ref, v_ref, bt_ref, lens_ref, out_ref,
             o_send, o_recv, st_send, st_recv, send_sems, recv_sems):
        my_x = lax.axis_index("x")
        my_y = lax.axis_index("y")
        my_z = lax.axis_index("z")
        partner = (my_x, my_y, 1 - my_z)

        barrier_sem = pltpu.get_barrier_semaphore()
        pl.semaphore_signal(barrier_sem, inc=1, device_id=partner,
                            device_id_type=pl.DeviceIdType.MESH)

        z_off = my_z * P_loc
        btv = bt_ref[:, :]
        lensv = lens_ref[:, :]
        p_iota = lax.broadcasted_iota(jnp.int32, (B, NB, P_loc), 2) + z_off
        j_iota = lax.broadcasted_iota(jnp.int32, (B, NB, P_loc), 1)
        hit = (btv[:, :, None] == p_iota) & (j_iota < lensv[:, :, None])
        w_page = jnp.sum(hit.astype(jnp.float32), axis=1)

        e_p = lax.broadcasted_iota(jnp.int32, (P_loc, KT), 0)
        e_k = lax.broadcasted_iota(jnp.int32, (P_loc, KT), 1)
        expand = (e_p == e_k // BS).astype(jnp.float32)
        w_tok = lax.dot_general(
            w_page, expand, (((1,), (0,)), ((), ())),
            preferred_element_type=jnp.float32)

        q = q_ref[:, 0, :, :]
        kf = k_ref[:, :, :, :].reshape(KT, H, D)
        vf = v_ref[:, :, :, :].reshape(KT, H, D)

        s = lax.dot_general(
            q, kf, (((2,), (2,)), ((1,), (1,))),
            preferred_element_type=jnp.float32) * scale
        neg = jnp.float32(-1e30)
        s = jnp.where(w_tok[None, :, :] > 0, s, neg)
        m = jnp.max(s, axis=2)
        p = jnp.exp(s - m[:, :, None]) * w_tok[None, :, :]
        l = jnp.sum(p, axis=2)
        o = lax.dot_general(
            p, vf, (((2,), (0,)), ((0,), (1,))),
            preferred_element_type=jnp.float32)

        o_send[:, :, :] = o
        st_send[0, :, :] = m
        st_send[1, :, :] = l

        pl.semaphore_wait(barrier_sem, 1)

        rdma_o = pltpu.make_async_remote_copy(
            src_ref=o_send, dst_ref=o_recv,
            send_sem=send_sems.at[0], recv_sem=recv_sems.at[0],
            device_id=partner, device_id_type=pl.DeviceIdType.MESH)
        rdma_st = pltpu.make_async_remote_copy(
            src_ref=st_send, dst_ref=st_recv,
            send_sem=send_sems.at[1], recv_sem=recv_sems.at[1],
            device_id=partner, device_id_type=pl.DeviceIdType.MESH)
        rdma_o.start()
        rdma_st.start()
        rdma_o.wait()
        rdma_st.wait()

        m_o = st_recv[0, :, :]
        l_o = st_recv[1, :, :]
        o_o = o_recv[:, :, :]
        m_n = jnp.maximum(m, m_o)
        a_l = jnp.exp(m - m_n)
        a_o = jnp.exp(m_o - m_n)
        l_tot = a_l * l + a_o * l_o
        comb = (a_l[:, :, None] * o + a_o[:, :, None] * o_o) \
            / l_tot[:, :, None]
        out_ref[:, 0, :, :] = jnp.transpose(comb, (1, 0, 2))

    return pl.pallas_call(
        body,
        out_shape=jax.ShapeDtypeStruct((B, QL, H, D), jnp.float32),
        in_specs=[pl.BlockSpec(memory_space=pltpu.VMEM)] * 5,
        out_specs=pl.BlockSpec(memory_space=pltpu.VMEM),
        scratch_shapes=[
            pltpu.VMEM((H, B, D), jnp.float32),
            pltpu.VMEM((H, B, D), jnp.float32),
            pltpu.VMEM((2, H, B), jnp.float32),
            pltpu.VMEM((2, H, B), jnp.float32),
            pltpu.SemaphoreType.DMA((2,)),
            pltpu.SemaphoreType.DMA((2,)),
        ],
        compiler_params=pltpu.CompilerParams(collective_id=0),
    )(Q, K, V, bt, lens.reshape(B, 1))


# baseline (device time: 71200 ns/iter reference)
import jax
import jax.numpy as jnp
from jax import lax
from jax.experimental import pallas as pl
from jax.experimental.pallas import tpu as pltpu


def kernel(Q, K, V, bt, lens):
    B, QL, H, D = Q.shape
    P_loc, BS, _, _ = K.shape
    NB = bt.shape[1]
    KT = P_loc * BS
    scale = D ** -0.5

    def body(q_ref, k_ref, v_ref, bt_ref, lens_ref, out_ref,
             o_send, o_recv, st_send, st_recv, send_sems, recv_sems):
        my_x = lax.axis_index("x")
        my_y = lax.axis_index("y")
        my_z = lax.axis_index("z")
        partner = (my_x, my_y, 1 - my_z)

        barrier_sem = pltpu.get_barrier_semaphore()
        pl.semaphore_signal(barrier_sem, inc=1, device_id=partner,
                            device_id_type=pl.DeviceIdType.MESH)

        z_off = my_z * P_loc
        btv = bt_ref[:, :]
        lensv = lens_ref[:, :]
        p_iota = lax.broadcasted_iota(jnp.int32, (B, NB, P_loc), 2) + z_off
        j_iota = lax.broadcasted_iota(jnp.int32, (B, NB, P_loc), 1)
        hit = (btv[:, :, None] == p_iota) & (j_iota < lensv[:, :, None])
        w_page = jnp.sum(hit.astype(jnp.float32), axis=1)

        e_p = lax.broadcasted_iota(jnp.int32, (P_loc, KT), 0)
        e_k = lax.broadcasted_iota(jnp.int32, (P_loc, KT), 1)
        expand = (e_p == e_k // BS).astype(jnp.float32)
        w_tok = lax.dot_general(
            w_page, expand, (((1,), (0,)), ((), ())),
            preferred_element_type=jnp.float32)
        w_pos = w_tok > 0
        neg = jnp.float32(-1e30)

        for h in range(H):
            q_h = q_ref[:, 0, h, :]
            k_h = k_ref[:, :, h, :].reshape(KT, D)
            v_h = v_ref[:, :, h, :].reshape(KT, D)
            s_h = lax.dot_general(
                q_h, k_h, (((1,), (1,)), ((), ())),
                preferred_element_type=jnp.float32) * scale
            s_h = jnp.where(w_pos, s_h, neg)
            m_h = jnp.max(s_h, axis=1, keepdims=True)
            p_h = jnp.exp(s_h - m_h) * w_tok
            l_h = jnp.sum(p_h, axis=1, keepdims=True)
            o_h = lax.dot_general(
                p_h, v_h, (((1,), (0,)), ((), ())),
                preferred_element_type=jnp.float32)
            o_send[h, :, :] = o_h
            st_send[h, :, 0:1] = m_h
            st_send[h, :, 1:2] = l_h

        pl.semaphore_wait(barrier_sem, 1)

        rdma_o = pltpu.make_async_remote_copy(
            src_ref=o_send, dst_ref=o_recv,
            send_sem=send_sems.at[0], recv_sem=recv_sems.at[0],
            device_id=partner, device_id_type=pl.DeviceIdType.MESH)
        rdma_st = pltpu.make_async_remote_copy(
            src_ref=st_send, dst_ref=st_recv,
            send_sem=send_sems.at[1], recv_sem=recv_sems.at[1],
            device_id=partner, device_id_type=pl.DeviceIdType.MESH)
        rdma_o.start()
        rdma_st.start()
        rdma_o.wait()
        rdma_st.wait()

        for h in range(H):
            m_l = st_send[h, :, 0:1]
            l_l = st_send[h, :, 1:2]
            m_r = st_recv[h, :, 0:1]
            l_r = st_recv[h, :, 1:2]
            m_n = jnp.maximum(m_l, m_r)
            a_l = jnp.exp(m_l - m_n)
            a_r = jnp.exp(m_r - m_n)
            l_tot = a_l * l_l + a_r * l_r
            comb = (a_l * o_send[h, :, :] + a_r * o_recv[h, :, :]) / l_tot
            out_ref[:, 0, h, :] = comb

    return pl.pallas_call(
        body,
        out_shape=jax.ShapeDtypeStruct((B, QL, H, D), jnp.float32),
        in_specs=[pl.BlockSpec(memory_space=pltpu.VMEM)] * 5,
        out_specs=pl.BlockSpec(memory_space=pltpu.VMEM),
        scratch_shapes=[
            pltpu.VMEM((H, B, D), jnp.float32),
            pltpu.VMEM((H, B, D), jnp.float32),
            pltpu.VMEM((H, B, 2), jnp.float32),
            pltpu.VMEM((H, B, 2), jnp.float32),
            pltpu.SemaphoreType.DMA((2,)),
            pltpu.SemaphoreType.DMA((2,)),
        ],
        compiler_params=pltpu.CompilerParams(collective_id=0),
    )(Q, K, V, bt, lens.reshape(B, 1))


# device time: 51451 ns/iter; 1.3838x vs baseline; 1.3838x over previous
import jax
import jax.numpy as jnp
from jax import lax
from jax.experimental import pallas as pl
from jax.experimental.pallas import tpu as pltpu


def kernel(Q, K, V, bt, lens):
    B, QL, H, D = Q.shape
    P_loc, BS, _, _ = K.shape
    NB = bt.shape[1]
    KT = P_loc * BS
    scale = D ** -0.5

    def body(q_ref, k_ref, v_ref, bt_ref, lens_ref, out_ref,
             kT, vT, o_send, o_recv, st_send, st_recv,
             dma_sems, send_sems, recv_sems):
        my_x = lax.axis_index("x")
        my_y = lax.axis_index("y")
        my_z = lax.axis_index("z")
        partner = (my_x, my_y, 1 - my_z)

        barrier_sem = pltpu.get_barrier_semaphore()
        pl.semaphore_signal(barrier_sem, inc=1, device_id=partner,
                            device_id_type=pl.DeviceIdType.MESH)

        k_dmas = []
        v_dmas = []
        for h in range(H):
            kd = pltpu.make_async_copy(
                k_ref.at[:, :, h, :], kT.at[h], dma_sems.at[0, h])
            vd = pltpu.make_async_copy(
                v_ref.at[:, :, h, :], vT.at[h], dma_sems.at[1, h])
            kd.start()
            vd.start()
            k_dmas.append(kd)
            v_dmas.append(vd)

        z_off = my_z * P_loc
        btv = bt_ref[:, :]
        lensv = lens_ref[:, :]
        p_iota = lax.broadcasted_iota(jnp.int32, (B, NB, P_loc), 2) + z_off
        j_iota = lax.broadcasted_iota(jnp.int32, (B, NB, P_loc), 1)
        hit = (btv[:, :, None] == p_iota) & (j_iota < lensv[:, :, None])
        w_page = jnp.sum(hit.astype(jnp.float32), axis=1)

        e_p = lax.broadcasted_iota(jnp.int32, (P_loc, KT), 0)
        e_k = lax.broadcasted_iota(jnp.int32, (P_loc, KT), 1)
        expand = (e_p == e_k // BS).astype(jnp.float32)
        w_tok = lax.dot_general(
            w_page, expand, (((1,), (0,)), ((), ())),
            preferred_element_type=jnp.float32)
        w_pos = w_tok > 0
        neg = jnp.float32(-1e30)

        for h in range(H):
            k_dmas[h].wait()
            v_dmas[h].wait()
            q_h = q_ref[:, 0, h, :]
            k_h = kT[h].reshape(KT, D)
            v_h = vT[h].reshape(KT, D)
            s_h = lax.dot_general(
                q_h, k_h, (((1,), (1,)), ((), ())),
                preferred_element_type=jnp.float32) * scale
            s_h = jnp.where(w_pos, s_h, neg)
            m_h = jnp.max(s_h, axis=1, keepdims=True)
            p_h = jnp.exp(s_h - m_h) * w_tok
            l_h = jnp.sum(p_h, axis=1, keepdims=True)
            o_h = lax.dot_general(
                p_h, v_h, (((1,), (0,)), ((), ())),
                preferred_element_type=jnp.float32)
            o_send[h, :, :] = o_h
            st_send[h, :, 0:1] = m_h
            st_send[h, :, 1:2] = l_h

        pl.semaphore_wait(barrier_sem, 1)

        rdma_o = pltpu.make_async_remote_copy(
            src_ref=o_send, dst_ref=o_recv,
            send_sem=send_sems.at[0], recv_sem=recv_sems.at[0],
            device_id=partner, device_id_type=pl.DeviceIdType.MESH)
        rdma_st = pltpu.make_async_remote_copy(
            src_ref=st_send, dst_ref=st_recv,
            send_sem=send_sems.at[1], recv_sem=recv_sems.at[1],
            device_id=partner, device_id_type=pl.DeviceIdType.MESH)
        rdma_o.start()
        rdma_st.start()
        rdma_o.wait()
        rdma_st.wait()

        for h in range(H):
            m_l = st_send[h, :, 0:1]
            l_l = st_send[h, :, 1:2]
            m_r = st_recv[h, :, 0:1]
            l_r = st_recv[h, :, 1:2]
            m_n = jnp.maximum(m_l, m_r)
            a_l = jnp.exp(m_l - m_n)
            a_r = jnp.exp(m_r - m_n)
            l_tot = a_l * l_l + a_r * l_r
            comb = (a_l * o_send[h, :, :] + a_r * o_recv[h, :, :]) / l_tot
            out_ref[:, 0, h, :] = comb

    return pl.pallas_call(
        body,
        out_shape=jax.ShapeDtypeStruct((B, QL, H, D), jnp.float32),
        in_specs=[
            pl.BlockSpec(memory_space=pltpu.VMEM),
            pl.BlockSpec(memory_space=pl.ANY),
            pl.BlockSpec(memory_space=pl.ANY),
            pl.BlockSpec(memory_space=pltpu.VMEM),
            pl.BlockSpec(memory_space=pltpu.VMEM),
        ],
        out_specs=pl.BlockSpec(memory_space=pltpu.VMEM),
        scratch_shapes=[
            pltpu.VMEM((H, P_loc, BS, D), jnp.float32),
            pltpu.VMEM((H, P_loc, BS, D), jnp.float32),
            pltpu.VMEM((H, B, D), jnp.float32),
            pltpu.VMEM((H, B, D), jnp.float32),
            pltpu.VMEM((H, B, 2), jnp.float32),
            pltpu.VMEM((H, B, 2), jnp.float32),
            pltpu.SemaphoreType.DMA((2, H)),
            pltpu.SemaphoreType.DMA((2,)),
            pltpu.SemaphoreType.DMA((2,)),
        ],
        compiler_params=pltpu.CompilerParams(
            collective_id=0, vmem_limit_bytes=48 * 1024 * 1024),
    )(Q, K, V, bt, lens.reshape(B, 1))


# device time: 25595 ns/iter; 2.7818x vs baseline; 2.0102x over previous
import jax
import jax.numpy as jnp
from jax import lax
from jax.experimental import pallas as pl
from jax.experimental.pallas import tpu as pltpu


def kernel(Q, K, V, bt, lens):
    B, QL, H, D = Q.shape
    P_loc, BS, _, _ = K.shape
    NB = bt.shape[1]
    scale = D ** -0.5

    def body(q_ref, k_ref, v_ref, bt_ref, lens_ref, out_ref,
             o_send, o_recv, st_send, st_recv, send_sems, recv_sems):
        my_x = lax.axis_index("x")
        my_y = lax.axis_index("y")
        my_z = lax.axis_index("z")
        partner = (my_x, my_y, 1 - my_z)

        barrier_sem = pltpu.get_barrier_semaphore()
        pl.semaphore_signal(barrier_sem, inc=1, device_id=partner,
                            device_id_type=pl.DeviceIdType.MESH)

        z_off = my_z * P_loc
        btv = bt_ref[:, :]
        lensv = lens_ref[:, :]
        p_iota = lax.broadcasted_iota(jnp.int32, (B, NB, P_loc), 2) + z_off
        j_iota = lax.broadcasted_iota(jnp.int32, (B, NB, P_loc), 1)
        hit = (btv[:, :, None] == p_iota) & (j_iota < lensv[:, :, None])
        w_page = jnp.sum(hit.astype(jnp.float32), axis=1)
        w_pos = w_page > 0
        neg = jnp.float32(-1e30)

        for h in range(H):
            q_h = q_ref[:, 0, h, :]
            k_h = k_ref[:, h, :, :]
            v_h = v_ref[:, h, :, :]
            qb = jnp.broadcast_to(q_h[None], (BS, B, D))
            s3 = lax.dot_general(
                qb, k_h, (((2,), (1,)), ((0,), (0,))),
                preferred_element_type=jnp.float32) * scale
            s3 = jnp.where(w_pos[None, :, :], s3, neg)
            m1 = jnp.max(s3, axis=0)
            m_h = jnp.max(m1, axis=1, keepdims=True)
            p3 = jnp.exp(s3 - m_h[None, :, :]) * w_page[None, :, :]
            l1 = jnp.sum(p3, axis=0)
            l_h = jnp.sum(l1, axis=1, keepdims=True)
            o3 = lax.dot_general(
                p3, v_h, (((2,), (2,)), ((0,), (0,))),
                preferred_element_type=jnp.float32)
            o_h = jnp.sum(o3, axis=0)
            o_send[h, :, :] = o_h
            st_send[h, :, 0:1] = m_h
            st_send[h, :, 1:2] = l_h

        pl.semaphore_wait(barrier_sem, 1)

        rdma_o = pltpu.make_async_remote_copy(
            src_ref=o_send, dst_ref=o_recv,
            send_sem=send_sems.at[0], recv_sem=recv_sems.at[0],
            device_id=partner, device_id_type=pl.DeviceIdType.MESH)
        rdma_st = pltpu.make_async_remote_copy(
            src_ref=st_send, dst_ref=st_recv,
            send_sem=send_sems.at[1], recv_sem=recv_sems.at[1],
            device_id=partner, device_id_type=pl.DeviceIdType.MESH)
        rdma_o.start()
        rdma_st.start()
        rdma_o.wait()
        rdma_st.wait()

        for h in range(H):
            m_l = st_send[h, :, 0:1]
            l_l = st_send[h, :, 1:2]
            m_r = st_recv[h, :, 0:1]
            l_r = st_recv[h, :, 1:2]
            m_n = jnp.maximum(m_l, m_r)
            a_l = jnp.exp(m_l - m_n)
            a_r = jnp.exp(m_r - m_n)
            l_tot = a_l * l_l + a_r * l_r
            comb = (a_l * o_send[h, :, :] + a_r * o_recv[h, :, :]) / l_tot
            out_ref[:, 0, h, :] = comb

    K2 = jnp.transpose(K, (1, 2, 3, 0))
    V2 = jnp.transpose(V, (1, 2, 3, 0))

    return pl.pallas_call(
        body,
        out_shape=jax.ShapeDtypeStruct((B, QL, H, D), jnp.float32),
        in_specs=[pl.BlockSpec(memory_space=pltpu.VMEM)] * 5,
        out_specs=pl.BlockSpec(memory_space=pltpu.VMEM),
        scratch_shapes=[
            pltpu.VMEM((H, B, D), jnp.float32),
            pltpu.VMEM((H, B, D), jnp.float32),
            pltpu.VMEM((H, B, 2), jnp.float32),
            pltpu.VMEM((H, B, 2), jnp.float32),
            pltpu.SemaphoreType.DMA((2,)),
            pltpu.SemaphoreType.DMA((2,)),
        ],
        compiler_params=pltpu.CompilerParams(
            collective_id=0, vmem_limit_bytes=48 * 1024 * 1024),
    )(Q, K2, V2, bt, lens.reshape(B, 1))
